# Initial kernel scaffold; baseline (speedup 1.0000x reference)
import jax
import jax.numpy as jnp
from jax.experimental import pallas as pl


def kernel(delta_norm, logstd, M_obs):
    raise NotImplementedError("write your pallas kernel here")


# speedup vs baseline: 1.0023x; 1.0023x over previous
import jax
import jax.numpy as jnp
from jax.experimental import pallas as pl
from jax.experimental.pallas import tpu as pltpu

_LANE = 128
_TILE_ROWS = 4096


def _loss_kernel(d_ref, s_ref, m_ref, out_ref, acc_l, acc_m, *, steps):
    j = pl.program_id(0)

    @pl.when(j == 0)
    def _():
        acc_l[...] = jnp.zeros_like(acc_l)
        acc_m[...] = jnp.zeros_like(acc_m)

    d = d_ref[...]
    s = s_ref[...]
    m = m_ref[...]
    l = (jnp.abs(d) + s) * m
    acc_l[...] += l.reshape(-1, 8, _LANE).sum(axis=0)
    acc_m[...] += m.reshape(-1, 8, _LANE).sum(axis=0)

    @pl.when(j == steps - 1)
    def _():
        sum_l = acc_l[...].sum()
        sum_m = acc_m[...].sum()
        out_ref[0, 0] = sum_l / sum_m


def kernel(delta_norm, logstd, M_obs):
    n = delta_norm.size
    chunk = _TILE_ROWS * _LANE
    steps = max(1, -(-n // chunk))
    padded = steps * chunk

    def prep(x):
        flat = jnp.reshape(x.astype(jnp.float32), (-1,))
        if padded != n:
            flat = jnp.pad(flat, (0, padded - n))
        return flat.reshape(steps * _TILE_ROWS, _LANE)

    d2, s2, m2 = prep(delta_norm), prep(logstd), prep(M_obs)

    tile_spec = pl.BlockSpec((_TILE_ROWS, _LANE), lambda j: (j, 0))
    out = pl.pallas_call(
        lambda *refs: _loss_kernel(*refs, steps=steps),
        out_shape=jax.ShapeDtypeStruct((1, 1), jnp.float32),
        grid=(steps,),
        in_specs=[tile_spec, tile_spec, tile_spec],
        out_specs=pl.BlockSpec(memory_space=pltpu.SMEM),
        scratch_shapes=[pltpu.VMEM((8, _LANE), jnp.float32),
                        pltpu.VMEM((8, _LANE), jnp.float32)],
        compiler_params=pltpu.CompilerParams(
            dimension_semantics=("arbitrary",)),
        cost_estimate=pl.CostEstimate(
            flops=int(5 * n), transcendentals=0,
            bytes_accessed=int(12 * n)),
    )(d2, s2, m2)
    return out[0, 0]


# speedup vs baseline: 3.0806x; 3.0736x over previous
import functools

import jax
import jax.numpy as jnp
from jax.experimental import pallas as pl
from jax.experimental.pallas import tpu as pltpu

_B = 8


def _loss_kernel(d_ref, s_ref, m_ref, out_ref, acc_l, acc_m, *, steps):
    j = pl.program_id(0)

    @pl.when(j == 0)
    def _():
        acc_l[...] = jnp.zeros_like(acc_l)
        acc_m[...] = jnp.zeros_like(acc_m)

    d = d_ref[...]
    s = s_ref[...]
    m = m_ref[...]
    l = (jnp.abs(d) + s) * m
    acc_l[...] += l.reshape(-1, 8, l.shape[-1]).sum(axis=0)
    acc_m[...] += m.reshape(-1, 8, m.shape[-1]).sum(axis=0)

    @pl.when(j == steps - 1)
    def _():
        out_ref[0, 0] = acc_l[...].sum() / acc_m[...].sum()


def kernel(delta_norm, logstd, M_obs):
    b0, b1, b2 = delta_norm.shape
    blk = _B
    while b0 % blk:
        blk //= 2
    steps = b0 // blk

    tile_spec = pl.BlockSpec((blk, b1, b2), lambda j: (j, 0, 0))
    out = pl.pallas_call(
        functools.partial(_loss_kernel, steps=steps),
        out_shape=jax.ShapeDtypeStruct((1, 1), jnp.float32),
        grid=(steps,),
        in_specs=[tile_spec, tile_spec, tile_spec],
        out_specs=pl.BlockSpec(memory_space=pltpu.SMEM),
        scratch_shapes=[pltpu.VMEM((8, b2), jnp.float32),
                        pltpu.VMEM((8, b2), jnp.float32)],
        compiler_params=pltpu.CompilerParams(
            dimension_semantics=("arbitrary",)),
        cost_estimate=pl.CostEstimate(
            flops=int(5 * delta_norm.size), transcendentals=0,
            bytes_accessed=int(12 * delta_norm.size)),
    )(delta_norm, logstd, M_obs)
    return out[0, 0]


# speedup vs baseline: 3.0824x; 1.0006x over previous
import functools

import jax
import jax.numpy as jnp
from jax.experimental import pallas as pl
from jax.experimental.pallas import tpu as pltpu

_TILE = 16384


def _sums_kernel(d_hbm, s_hbm, m_hbm, out_ref, d_buf, s_buf, m_buf,
                 acc_l, acc_m, sem, *, rows, tile, minor):
    steps = rows // tile
    rem = rows - steps * tile
    hbms = (d_hbm.reshape(rows, minor), s_hbm.reshape(rows, minor),
            m_hbm.reshape(rows, minor))
    bufs = (d_buf, s_buf, m_buf)

    def dma(slot, step, nrows):
        for k in range(3):
            pltpu.make_async_copy(
                hbms[k].at[pl.ds(step * tile, nrows)],
                bufs[k].at[slot, pl.ds(0, nrows)],
                sem.at[k, slot]).start()

    def wait(slot, nrows):
        for k in range(3):
            pltpu.make_async_copy(
                hbms[k].at[pl.ds(0, nrows)],
                bufs[k].at[slot, pl.ds(0, nrows)],
                sem.at[k, slot]).wait()

    def accumulate(d, s, m):
        l = (jnp.abs(d) + s) * m
        acc_l[...] += l.reshape(-1, 8, minor).sum(axis=0)
        acc_m[...] += m.reshape(-1, 8, minor).sum(axis=0)

    acc_l[...] = jnp.zeros_like(acc_l)
    acc_m[...] = jnp.zeros_like(acc_m)

    if steps > 0:
        dma(0, 0, tile)

        def body(j, _):
            cur = jax.lax.rem(j, 2)
            nxt = jax.lax.rem(j + 1, 2)

            @pl.when(j + 1 < steps)
            def _():
                dma(nxt, j + 1, tile)

            wait(cur, tile)
            accumulate(d_buf[cur], s_buf[cur], m_buf[cur])
            return ()

        jax.lax.fori_loop(0, steps, body, (), unroll=False)

    if rem:
        tslot = steps % 2
        dma(tslot, steps, rem)
        wait(tslot, rem)
        d = d_buf[tslot, :rem]
        s = s_buf[tslot, :rem]
        m = m_buf[tslot, :rem]
        pad = (-rem) % 8
        if pad:
            z = jnp.zeros((pad, minor), jnp.float32)
            d = jnp.concatenate([d, z], 0)
            s = jnp.concatenate([s, z], 0)
            m = jnp.concatenate([m, z], 0)
        accumulate(d, s, m)

    out_ref[0, 0] = acc_l[...].sum() / acc_m[...].sum()


def kernel(delta_norm, logstd, M_obs):
    f32 = jnp.float32
    if delta_norm.ndim < 2:
        delta_norm = delta_norm.reshape(1, -1)
        logstd = logstd.reshape(1, -1)
        M_obs = M_obs.reshape(1, -1)
    shape = delta_norm.shape
    minor = shape[-1]
    rows = delta_norm.size // minor
    tile = min(_TILE, max(8, (rows // 8) * 8))

    out = pl.pallas_call(
        functools.partial(_sums_kernel, rows=rows, tile=tile, minor=minor),
        out_shape=jax.ShapeDtypeStruct((1, 1), f32),
        in_specs=[pl.BlockSpec(memory_space=pltpu.MemorySpace.HBM)] * 3,
        out_specs=pl.BlockSpec(memory_space=pltpu.SMEM),
        scratch_shapes=[pltpu.VMEM((2, tile, minor), f32),
                        pltpu.VMEM((2, tile, minor), f32),
                        pltpu.VMEM((2, tile, minor), f32),
                        pltpu.VMEM((8, minor), f32),
                        pltpu.VMEM((8, minor), f32),
                        pltpu.SemaphoreType.DMA((3, 2))],
        cost_estimate=pl.CostEstimate(
            flops=int(5 * delta_norm.size), transcendentals=0,
            bytes_accessed=int(12 * delta_norm.size)),
    )(delta_norm, logstd, M_obs)
    return out[0, 0]
